# Initial kernel scaffold; baseline (speedup 1.0000x reference)
import functools

import jax
import jax.numpy as jnp
from jax import lax
from jax.experimental import pallas as pl
from jax.experimental.pallas import tpu as pltpu
from jax.experimental.pallas import tpu_sc as plsc

D_EMBD = 32
B_TOTAL = 16384 * 50

_info = plsc.get_sparse_core_info()
_NC = _info.num_cores
_NS = _info.num_subcores
_NW = _NC * _NS
_B_PER_W = B_TOTAL // _NW
_CHUNK = 3200
_N_CHUNKS = _B_PER_W // _CHUNK

_mesh = plsc.VectorSubcoreMesh(core_axis_name="c", subcore_axis_name="s")


@functools.partial(
    pl.kernel,
    mesh=_mesh,
    out_type=jax.ShapeDtypeStruct((B_TOTAL, D_EMBD), jnp.float32),
    scratch_types=[
        pltpu.VMEM((_CHUNK,), jnp.int32),
        pltpu.VMEM((_CHUNK, D_EMBD), jnp.float32),
        pltpu.SemaphoreType.DMA,
    ],
)
def _gather(idx_hbm, tab_hbm, out_hbm, idx_v, rows_v, sem):
    wid = lax.axis_index("s") * _NC + lax.axis_index("c")
    base = wid * _B_PER_W
    for j in range(_N_CHUNKS):
        off = base + j * _CHUNK
        pltpu.sync_copy(idx_hbm.at[pl.ds(off, _CHUNK)], idx_v)
        pltpu.async_copy(tab_hbm.at[idx_v], rows_v, sem).wait()
        pltpu.sync_copy(rows_v, out_hbm.at[pl.ds(off, _CHUNK)])


def kernel(x, embd):
    flat_idx = x.reshape(-1).astype(jnp.int32)
    out = _gather(flat_idx, embd)
    return out.reshape(x.shape[0], x.shape[1], D_EMBD)


# speedup vs baseline: 1.0091x; 1.0091x over previous
import functools

import jax
import jax.numpy as jnp
from jax import lax
from jax.experimental import pallas as pl
from jax.experimental.pallas import tpu as pltpu
from jax.experimental.pallas import tpu_sc as plsc

D_EMBD = 32
B_TOTAL = 16384 * 50

_info = plsc.get_sparse_core_info()
_NC = _info.num_cores
_NS = _info.num_subcores
_NW = _NC * _NS
_B_PER_W = B_TOTAL // _NW
_CHUNK = 3200
_N_CHUNKS = _B_PER_W // _CHUNK

_mesh = plsc.VectorSubcoreMesh(core_axis_name="c", subcore_axis_name="s")


@functools.partial(
    pl.kernel,
    mesh=_mesh,
    out_type=jax.ShapeDtypeStruct((B_TOTAL, D_EMBD), jnp.float32),
    scratch_types=[
        pltpu.VMEM((_CHUNK,), jnp.int32),
        pltpu.VMEM((_CHUNK, D_EMBD), jnp.float32),
        pltpu.SemaphoreType.DMA,
    ],
    compiler_params=pltpu.CompilerParams(use_tc_tiling_on_sc=False),
)
def _gather(idx_hbm, tab_hbm, out_hbm, idx_v, rows_v, sem):
    wid = lax.axis_index("s") * _NC + lax.axis_index("c")
    base = wid * _B_PER_W
    for j in range(_N_CHUNKS):
        off = base + j * _CHUNK
        pltpu.sync_copy(idx_hbm.at[pl.ds(off, _CHUNK)], idx_v)
        pltpu.async_copy(tab_hbm.at[idx_v], rows_v, sem).wait()
        pltpu.sync_copy(rows_v, out_hbm.at[pl.ds(off, _CHUNK)])


def kernel(x, embd):
    flat_idx = x.reshape(-1).astype(jnp.int32)
    out = _gather(flat_idx, embd)
    return out.reshape(x.shape[0], x.shape[1], D_EMBD)


# speedup vs baseline: 1.0099x; 1.0009x over previous
import functools

import jax
import jax.numpy as jnp
from jax import lax
from jax.experimental import pallas as pl
from jax.experimental.pallas import tpu as pltpu
from jax.experimental.pallas import tpu_sc as plsc

D_EMBD = 32
B_TOTAL = 16384 * 50

_info = plsc.get_sparse_core_info()
_NC = _info.num_cores
_NS = _info.num_subcores
_NW = _NC * _NS
_B_PER_W = B_TOTAL // _NW
_CHUNK = 1600
_N_CHUNKS = _B_PER_W // _CHUNK

_mesh = plsc.VectorSubcoreMesh(core_axis_name="c", subcore_axis_name="s")


@functools.partial(
    pl.kernel,
    mesh=_mesh,
    out_type=jax.ShapeDtypeStruct((B_TOTAL, D_EMBD), jnp.float32),
    scratch_types=[
        pltpu.VMEM((_CHUNK,), jnp.int32),
        pltpu.VMEM((_CHUNK,), jnp.int32),
        pltpu.VMEM((_CHUNK, D_EMBD), jnp.float32),
        pltpu.VMEM((_CHUNK, D_EMBD), jnp.float32),
        pltpu.SemaphoreType.DMA,
        pltpu.SemaphoreType.DMA,
        pltpu.SemaphoreType.DMA,
        pltpu.SemaphoreType.DMA,
        pltpu.SemaphoreType.DMA,
        pltpu.SemaphoreType.DMA,
    ],
    compiler_params=pltpu.CompilerParams(use_tc_tiling_on_sc=False),
)
def _gather(idx_hbm, tab_hbm, out_hbm,
            idx_v0, idx_v1, rows_v0, rows_v1,
            si0, si1, sg0, sg1, so0, so1):
    wid = lax.axis_index("s") * _NC + lax.axis_index("c")
    base = wid * _B_PER_W
    idx_v = (idx_v0, idx_v1)
    rows_v = (rows_v0, rows_v1)
    sem_i = (si0, si1)
    sem_g = (sg0, sg1)
    sem_o = (so0, so1)

    def idx_off(j):
        return pl.ds(base + j * _CHUNK, _CHUNK)

    ih = [None, None]
    gh = [None, None]
    oh = [None, None]
    ih[0] = pltpu.async_copy(idx_hbm.at[idx_off(0)], idx_v[0], sem_i[0])
    ih[1] = pltpu.async_copy(idx_hbm.at[idx_off(1)], idx_v[1], sem_i[1])
    for j in range(_N_CHUNKS):
        b = j & 1
        if oh[b] is not None:
            oh[b].wait()
        ih[b].wait()
        gh[b] = pltpu.async_copy(tab_hbm.at[idx_v[b]], rows_v[b], sem_g[b])
        if j >= 1:
            gh[1 - b].wait()
            oh[1 - b] = pltpu.async_copy(
                rows_v[1 - b], out_hbm.at[idx_off(j - 1)], sem_o[1 - b])
            if j + 1 < _N_CHUNKS:
                ih[1 - b] = pltpu.async_copy(
                    idx_hbm.at[idx_off(j + 1)], idx_v[1 - b], sem_i[1 - b])
    bl = (_N_CHUNKS - 1) & 1
    gh[bl].wait()
    oh[bl] = pltpu.async_copy(
        rows_v[bl], out_hbm.at[idx_off(_N_CHUNKS - 1)], sem_o[bl])
    oh[1 - bl].wait()
    oh[bl].wait()


def kernel(x, embd):
    flat_idx = x.reshape(-1).astype(jnp.int32)
    out = _gather(flat_idx, embd)
    return out.reshape(x.shape[0], x.shape[1], D_EMBD)
